# baseline (speedup 1.0000x reference)
import functools

import jax
import jax.numpy as jnp
from jax import lax
from jax.experimental import pallas as pl
from jax.experimental.pallas import tpu as pltpu
from jax.experimental.pallas import tpu_sc as plsc

_NUM_EXPERTS = 64
_TOP_K = 8
_BLOCK_T = 1024


def _router_block(x_ref, w_ref, b_ref, out_ref):
    x = x_ref[...]
    w = w_ref[...]
    b = b_ref[...]
    logits = jnp.dot(x, w, preferred_element_type=jnp.float32) + b[None, :]

    lt = logits.T
    rows = lt.shape[1]
    e_iota = lax.broadcasted_iota(jnp.int32, (_NUM_EXPERTS, rows), 0)

    remaining = lt
    mask = jnp.zeros((_NUM_EXPERTS, rows), dtype=jnp.bool_)
    row_max = None
    for k in range(_TOP_K):
        m = jnp.max(remaining, axis=0, keepdims=True)
        if k == 0:
            row_max = m
        cand = jnp.where(remaining == m, e_iota, _NUM_EXPERTS)
        sel = jnp.min(cand, axis=0, keepdims=True)
        sel_mask = e_iota == sel
        mask = mask | sel_mask
        remaining = jnp.where(sel_mask, -jnp.inf, remaining)

    expw = jnp.where(mask, jnp.exp(lt - row_max), 0.0)
    inv = 1.0 / jnp.sum(expw, axis=0, keepdims=True)
    out_ref[...] = (expw * inv).T


def _fused_tc_kernel(hidden_states, W, b):
    tokens, d_model = hidden_states.shape
    grid = (tokens // _BLOCK_T,)
    return pl.pallas_call(
        _router_block,
        grid=grid,
        in_specs=[
            pl.BlockSpec((_BLOCK_T, d_model), lambda i: (i, 0)),
            pl.BlockSpec((d_model, _NUM_EXPERTS), lambda i: (0, 0)),
            pl.BlockSpec((_NUM_EXPERTS,), lambda i: (0,)),
        ],
        out_specs=pl.BlockSpec((_BLOCK_T, _NUM_EXPERTS), lambda i: (i, 0)),
        out_shape=jax.ShapeDtypeStruct((tokens, _NUM_EXPERTS), jnp.float32),
    )(hidden_states, W, b)


def _logits_t_block(x_ref, w_ref, b_ref, out_ref):
    logits = (
        jnp.dot(x_ref[...], w_ref[...], preferred_element_type=jnp.float32)
        + b_ref[...][None, :]
    )
    out_ref[...] = logits.T


def _tc_logits_t(hidden_states, W, b):
    tokens, d_model = hidden_states.shape
    grid = (tokens // _BLOCK_T,)
    return pl.pallas_call(
        _logits_t_block,
        grid=grid,
        in_specs=[
            pl.BlockSpec((_BLOCK_T, d_model), lambda i: (i, 0)),
            pl.BlockSpec((d_model, _NUM_EXPERTS), lambda i: (0, 0)),
            pl.BlockSpec((_NUM_EXPERTS,), lambda i: (0,)),
        ],
        out_specs=pl.BlockSpec((_NUM_EXPERTS, _BLOCK_T), lambda i: (0, i)),
        out_shape=jax.ShapeDtypeStruct((_NUM_EXPERTS, tokens), jnp.float32),
    )(hidden_states, W, b)


def _sc_topk_softmax_t(logits_t):
    tokens = logits_t.shape[1]
    info = plsc.get_sparse_core_info()
    num_workers = info.num_cores * info.num_subcores
    tpw = tokens // num_workers
    lanes = 16
    mesh = plsc.VectorSubcoreMesh(core_axis_name="c", subcore_axis_name="s")

    @functools.partial(
        pl.kernel,
        mesh=mesh,
        out_type=jax.ShapeDtypeStruct((_NUM_EXPERTS, tokens), jnp.float32),
        scratch_types=[
            pltpu.VMEM((_NUM_EXPERTS, tpw), jnp.float32),
            pltpu.VMEM((_NUM_EXPERTS, tpw), jnp.float32),
        ],
    )
    def sc_kernel(logits_hbm, out_hbm, in_v, out_v):
        wid = lax.axis_index("s") * info.num_cores + lax.axis_index("c")
        base = wid * tpw
        pltpu.sync_copy(logits_hbm.at[:, pl.ds(base, tpw)], in_v)

        def group_body(g, carry):
            sl = pl.ds(g * lanes, lanes)
            vals = [in_v[e, sl] for e in range(_NUM_EXPERTS)]

            def tree_reduce(op, xs):
                while len(xs) > 1:
                    nxt = [op(xs[i], xs[i + 1]) for i in range(0, len(xs) - 1, 2)]
                    if len(xs) % 2:
                        nxt.append(xs[-1])
                    xs = nxt
                return xs[0]

            rem = list(vals)
            selected = [None] * _NUM_EXPERTS
            row_max = None
            for k in range(_TOP_K):
                m = tree_reduce(jnp.maximum, rem)
                if k == 0:
                    row_max = m
                idx = jnp.full((lanes,), _NUM_EXPERTS, jnp.int32)
                for e in range(_NUM_EXPERTS - 1, -1, -1):
                    idx = jnp.where(rem[e] == m, e, idx)
                neg = jnp.float32(-jnp.inf)
                for e in range(_NUM_EXPERTS):
                    hit = idx == e
                    selected[e] = hit if selected[e] is None else (selected[e] | hit)
                    rem[e] = jnp.where(hit, neg, rem[e])

            w = [
                jnp.where(selected[e], jnp.exp(vals[e] - row_max), 0.0)
                for e in range(_NUM_EXPERTS)
            ]
            inv = 1.0 / tree_reduce(jnp.add, w)
            for e in range(_NUM_EXPERTS):
                out_v[e, sl] = w[e] * inv
            return carry

        lax.fori_loop(0, tpw // lanes, group_body, 0)
        pltpu.sync_copy(out_v, out_hbm.at[:, pl.ds(base, tpw)])

    return sc_kernel(logits_t)


def _hybrid_sc_kernel(hidden_states, W, b):
    scores_t = _sc_topk_softmax_t(_tc_logits_t(hidden_states, W, b))
    return scores_t.T


@functools.partial(jax.jit, static_argnames=())
def kernel(hidden_states, W, b):
    return _hybrid_sc_kernel(hidden_states, W, b)


# speedup vs baseline: 2.7393x; 2.7393x over previous
import functools

import jax
import jax.numpy as jnp
from jax import lax
from jax.experimental import pallas as pl
from jax.experimental.pallas import tpu as pltpu
from jax.experimental.pallas import tpu_sc as plsc

_NUM_EXPERTS = 64
_TOP_K = 8
_BLOCK_T = 1024


def _router_block(x_ref, w_ref, b_ref, out_ref):
    x = x_ref[...]
    w = w_ref[...]
    b = b_ref[...]
    logits = jnp.dot(x, w, preferred_element_type=jnp.float32) + b[None, :]

    lt = logits.T
    rows = lt.shape[1]
    e_iota = lax.broadcasted_iota(jnp.int32, (_NUM_EXPERTS, rows), 0)

    remaining = lt
    mask = jnp.zeros((_NUM_EXPERTS, rows), dtype=jnp.bool_)
    row_max = None
    for k in range(_TOP_K):
        m = jnp.max(remaining, axis=0, keepdims=True)
        if k == 0:
            row_max = m
        cand = jnp.where(remaining == m, e_iota, _NUM_EXPERTS)
        sel = jnp.min(cand, axis=0, keepdims=True)
        sel_mask = e_iota == sel
        mask = mask | sel_mask
        remaining = jnp.where(sel_mask, -jnp.inf, remaining)

    expw = jnp.where(mask, jnp.exp(lt - row_max), 0.0)
    inv = 1.0 / jnp.sum(expw, axis=0, keepdims=True)
    out_ref[...] = (expw * inv).T


def _fused_tc_kernel(hidden_states, W, b):
    tokens, d_model = hidden_states.shape
    grid = (tokens // _BLOCK_T,)
    return pl.pallas_call(
        _router_block,
        grid=grid,
        in_specs=[
            pl.BlockSpec((_BLOCK_T, d_model), lambda i: (i, 0)),
            pl.BlockSpec((d_model, _NUM_EXPERTS), lambda i: (0, 0)),
            pl.BlockSpec((_NUM_EXPERTS,), lambda i: (0,)),
        ],
        out_specs=pl.BlockSpec((_BLOCK_T, _NUM_EXPERTS), lambda i: (i, 0)),
        out_shape=jax.ShapeDtypeStruct((tokens, _NUM_EXPERTS), jnp.float32),
    )(hidden_states, W, b)


def _logits_t_block(x_ref, w_ref, b_ref, out_ref):
    logits = (
        jnp.dot(x_ref[...], w_ref[...], preferred_element_type=jnp.float32)
        + b_ref[...][None, :]
    )
    out_ref[...] = logits.T


def _tc_logits_t(hidden_states, W, b):
    tokens, d_model = hidden_states.shape
    grid = (tokens // _BLOCK_T,)
    return pl.pallas_call(
        _logits_t_block,
        grid=grid,
        in_specs=[
            pl.BlockSpec((_BLOCK_T, d_model), lambda i: (i, 0)),
            pl.BlockSpec((d_model, _NUM_EXPERTS), lambda i: (0, 0)),
            pl.BlockSpec((_NUM_EXPERTS,), lambda i: (0,)),
        ],
        out_specs=pl.BlockSpec((_NUM_EXPERTS, _BLOCK_T), lambda i: (0, i)),
        out_shape=jax.ShapeDtypeStruct((_NUM_EXPERTS, tokens), jnp.float32),
    )(hidden_states, W, b)


def _sc_topk_softmax_t(logits_t):
    tokens = logits_t.shape[1]
    info = plsc.get_sparse_core_info()
    num_workers = info.num_cores * info.num_subcores
    tpw = tokens // num_workers
    lanes = 16
    mesh = plsc.VectorSubcoreMesh(core_axis_name="c", subcore_axis_name="s")

    @functools.partial(
        pl.kernel,
        mesh=mesh,
        out_type=jax.ShapeDtypeStruct((_NUM_EXPERTS, tokens), jnp.float32),
        scratch_types=[
            pltpu.VMEM((_NUM_EXPERTS, tpw), jnp.float32),
            pltpu.VMEM((_NUM_EXPERTS, tpw), jnp.float32),
        ],
    )
    def sc_kernel(logits_hbm, out_hbm, in_v, out_v):
        wid = lax.axis_index("s") * info.num_cores + lax.axis_index("c")
        base = wid * tpw
        pltpu.sync_copy(logits_hbm.at[:, pl.ds(base, tpw)], in_v)

        def group_body(g, carry):
            sl = pl.ds(g * lanes, lanes)
            vals = [in_v[e, sl] for e in range(_NUM_EXPERTS)]

            def tree_reduce(op, xs):
                while len(xs) > 1:
                    nxt = [op(xs[i], xs[i + 1]) for i in range(0, len(xs) - 1, 2)]
                    if len(xs) % 2:
                        nxt.append(xs[-1])
                    xs = nxt
                return xs[0]

            rem = list(vals)
            selected = [None] * _NUM_EXPERTS
            row_max = None
            for k in range(_TOP_K):
                m = tree_reduce(jnp.maximum, rem)
                if k == 0:
                    row_max = m
                idx = jnp.full((lanes,), _NUM_EXPERTS, jnp.int32)
                for e in range(_NUM_EXPERTS - 1, -1, -1):
                    idx = jnp.where(rem[e] == m, e, idx)
                neg = jnp.float32(-jnp.inf)
                for e in range(_NUM_EXPERTS):
                    hit = idx == e
                    selected[e] = hit if selected[e] is None else (selected[e] | hit)
                    rem[e] = jnp.where(hit, neg, rem[e])

            w = [
                jnp.where(selected[e], jnp.exp(vals[e] - row_max), 0.0)
                for e in range(_NUM_EXPERTS)
            ]
            inv = 1.0 / tree_reduce(jnp.add, w)
            for e in range(_NUM_EXPERTS):
                out_v[e, sl] = w[e] * inv
            return carry

        lax.fori_loop(0, tpw // lanes, group_body, 0)
        pltpu.sync_copy(out_v, out_hbm.at[:, pl.ds(base, tpw)])

    return sc_kernel(logits_t)


def _hybrid_sc_kernel(hidden_states, W, b):
    scores_t = _sc_topk_softmax_t(_tc_logits_t(hidden_states, W, b))
    return scores_t.T


@functools.partial(jax.jit, static_argnames=())
def kernel(hidden_states, W, b):
    return _fused_tc_kernel(hidden_states, W, b)
